# baseline (speedup 1.0000x reference)
import functools

import jax
import jax.numpy as jnp
from jax import lax
from jax.experimental import pallas as pl
from jax.experimental.pallas import tpu as pltpu
from jax.experimental.pallas import tpu_sc as plsc

N = 10000
D = 128
H = 16
CHUNK = 128
NBUF = 4
BLK = 4096


def _tables_body(h_ref, wof_ref, bof_ref, wnb_ref, bnb_ref, tof_ref, tnb_ref):
    h = h_ref[...]
    tof_ref[...] = jnp.dot(h, wof_ref[...], preferred_element_type=jnp.float32) + bof_ref[...]
    tnb_ref[...] = jnp.dot(h, wnb_ref[...], preferred_element_type=jnp.float32) + bnb_ref[...]


def _make_tables(h, wof, bof, wnb, bnb):
    return pl.pallas_call(
        _tables_body,
        out_shape=[
            jax.ShapeDtypeStruct((N, 2 * H), jnp.float32),
            jax.ShapeDtypeStruct((N, 2 * H), jnp.float32),
        ],
    )(h, wof, bof, wnb, bnb)


def _make_gather(ptc_of, ptc_nb):
    info = plsc.get_sparse_core_info()
    nc, ns = info.num_cores, info.num_subcores
    nw = nc * ns
    ppad_of = ptc_of * nw * CHUNK
    ppad_nb = ptc_nb * nw * CHUNK
    ptot = ppad_of + ppad_nb
    ptc_max = max(ptc_of, ptc_nb)
    mesh = plsc.VectorSubcoreMesh(core_axis_name="c", subcore_axis_name="s")

    @functools.partial(
        pl.kernel,
        mesh=mesh,
        out_type=jax.ShapeDtypeStruct((ptot, H), jnp.float32),
        scratch_types=[
            pltpu.VMEM((ptc_max, CHUNK), jnp.int32),
            pltpu.VMEM((ptc_max, CHUNK), jnp.int32),
            pltpu.VMEM((NBUF, CHUNK, 2 * H), jnp.float32),
            pltpu.VMEM((NBUF, CHUNK, 2 * H), jnp.float32),
            pltpu.VMEM((NBUF, CHUNK, H), jnp.float32),
            pltpu.SemaphoreType.DMA((NBUF,)),
            pltpu.SemaphoreType.DMA((NBUF,)),
            pltpu.SemaphoreType.DMA((NBUF,)),
        ],
        compiler_params=pltpu.CompilerParams(use_tc_tiling_on_sc=False),
    )
    def k(tof_hbm, i0of_hbm, i1of_hbm, tnb_hbm, i0nb_hbm, i1nb_hbm, s_hbm,
          idx0_v, idx1_v, rows0, rows1, s_buf, gsem0, gsem1, wsem):
        wid = lax.axis_index("s") * nc + lax.axis_index("c")

        def run_level(t_hbm, i0_hbm, i1_hbm, ptc, out_base):
            crow0 = wid * ptc
            pltpu.sync_copy(i0_hbm.at[pl.ds(crow0, ptc)], idx0_v.at[pl.ds(0, ptc)])
            pltpu.sync_copy(i1_hbm.at[pl.ds(crow0, ptc)], idx1_v.at[pl.ds(0, ptc)])

            def fire_g(c, b):
                pltpu.async_copy(t_hbm.at[idx0_v.at[c]], rows0.at[b], gsem0.at[b])
                pltpu.async_copy(t_hbm.at[idx1_v.at[c]], rows1.at[b], gsem1.at[b])

            def wait_g(c, b):
                pltpu.make_async_copy(t_hbm.at[idx0_v.at[c]], rows0.at[b], gsem0.at[b]).wait()
                pltpu.make_async_copy(t_hbm.at[idx1_v.at[c]], rows1.at[b], gsem1.at[b]).wait()

            def compute(b):
                def srow(i, carry):
                    lo0 = rows0[b, i, 0:H]
                    hi0 = rows0[b, i, H:2 * H]
                    lo1 = rows1[b, i, 0:H]
                    hi1 = rows1[b, i, H:2 * H]
                    s_buf[b, i, :] = (jnp.maximum(lo0 + hi1, 0.0)
                                      + jnp.maximum(lo1 + hi0, 0.0))
                    return carry
                lax.fori_loop(0, CHUNK, srow, 0)

            def fire_w(c, b):
                dst = pl.ds(out_base + (crow0 + c) * CHUNK, CHUNK)
                pltpu.async_copy(s_buf.at[b], s_hbm.at[dst], wsem.at[b])

            def wait_w(b):
                pltpu.make_async_copy(s_buf.at[b], s_hbm.at[pl.ds(0, CHUNK)],
                                      wsem.at[b]).wait()

            ngroups = ptc // NBUF
            for b in range(NBUF):
                fire_g(b, b)
            for b in range(NBUF):
                wait_g(b, b)
                compute(b)
                fire_w(b, b)
                fire_g(b + NBUF, b)

            def body(g, carry):
                for b in range(NBUF):
                    c = NBUF * g + b
                    wait_g(c, b)
                    wait_w(b)
                    compute(b)
                    fire_w(c, b)
                    fire_g(c + NBUF, b)
                return carry

            lax.fori_loop(1, ngroups - 1, body, 0)
            for b in range(NBUF):
                c = NBUF * (ngroups - 1) + b
                wait_g(c, b)
                wait_w(b)
                compute(b)
                fire_w(c, b)
            for b in range(NBUF):
                wait_w(b)

        run_level(tof_hbm, i0of_hbm, i1of_hbm, ptc_of, 0)
        run_level(tnb_hbm, i0nb_hbm, i1nb_hbm, ptc_nb, ppad_of)

    return k


def _finish(s, wh_of, bh_of, wh_nb, bh_nb, ppad_of, ptot):
    blocks_of = ppad_of // BLK

    def body(s_ref, whof_ref, bhof_ref, whnb_ref, bhnb_ref, out_ref):
        is_of = pl.program_id(0) < blocks_of
        wh = jnp.where(is_of, whof_ref[...], whnb_ref[...])
        bh = jnp.where(is_of, bhof_ref[...], bhnb_ref[...])
        out_ref[...] = jnp.dot(s_ref[...], wh, preferred_element_type=jnp.float32) + bh

    return pl.pallas_call(
        body,
        grid=(ptot // BLK,),
        in_specs=[
            pl.BlockSpec((BLK, H), lambda i: (i, 0)),
            pl.BlockSpec((H, 2), lambda i: (0, 0)),
            pl.BlockSpec((1, 2), lambda i: (0, 0)),
            pl.BlockSpec((H, 2), lambda i: (0, 0)),
            pl.BlockSpec((1, 2), lambda i: (0, 0)),
        ],
        out_specs=pl.BlockSpec((BLK, 2), lambda i: (i, 0)),
        out_shape=jax.ShapeDtypeStruct((ptot, 2), jnp.float32),
    )(s, wh_of, bh_of, wh_nb, bh_nb)


def _pad_idx(idx, ppad):
    p = idx.shape[0]
    if p != ppad:
        idx = jnp.concatenate([idx, jnp.zeros((ppad - p,), jnp.int32)])
    return idx.reshape(ppad // CHUNK, CHUNK)


def kernel(h, idx0_onefour, idx1_onefour, idx0_nonbonded, idx1_nonbonded,
           W1_of, b1_of, Wsig_of, bsig_of, Weps_of, beps_of,
           W1_nb, b1_nb, Wsig_nb, bsig_nb, Weps_nb, beps_nb):
    wof = jnp.concatenate([W1_of[:D], W1_of[D:]], axis=1)
    wnb = jnp.concatenate([W1_nb[:D], W1_nb[D:]], axis=1)
    bof = jnp.concatenate([jnp.zeros((H,), jnp.float32), b1_of]).reshape(1, 2 * H)
    bnb = jnp.concatenate([jnp.zeros((H,), jnp.float32), b1_nb]).reshape(1, 2 * H)
    wh_of = jnp.concatenate([Wsig_of, Weps_of], axis=1)
    wh_nb = jnp.concatenate([Wsig_nb, Weps_nb], axis=1)
    bh_of = jnp.concatenate([bsig_of, beps_of]).reshape(1, 2)
    bh_nb = jnp.concatenate([bsig_nb, beps_nb]).reshape(1, 2)

    t_of, t_nb = _make_tables(h, wof, bof, wnb, bnb)

    p_of = idx0_onefour.shape[0]
    p_nb = idx0_nonbonded.shape[0]
    gran = 32 * CHUNK * NBUF * 2
    ppad_of = ((p_of + gran - 1) // gran) * gran
    ppad_nb = ((p_nb + gran - 1) // gran) * gran

    s = _make_gather(ppad_of // (32 * CHUNK), ppad_nb // (32 * CHUNK))(
        t_of, _pad_idx(idx0_onefour, ppad_of), _pad_idx(idx1_onefour, ppad_of),
        t_nb, _pad_idx(idx0_nonbonded, ppad_nb), _pad_idx(idx1_nonbonded, ppad_nb))

    out = _finish(s, wh_of, bh_of, wh_nb, bh_nb, ppad_of, ppad_of + ppad_nb)
    return (out[:p_of], out[ppad_of:ppad_of + p_nb])


# speedup vs baseline: 1.0287x; 1.0287x over previous
import functools

import jax
import jax.numpy as jnp
from jax import lax
from jax.experimental import pallas as pl
from jax.experimental.pallas import tpu as pltpu
from jax.experimental.pallas import tpu_sc as plsc

N = 10000
D = 128
H = 16
CHUNK = 128
NBUF = 2
BLK = 4096


def _tables_body(h_ref, wof_ref, bof_ref, wnb_ref, bnb_ref, tof_ref, tnb_ref):
    h = h_ref[...]
    tof_ref[...] = jnp.dot(h, wof_ref[...], preferred_element_type=jnp.float32) + bof_ref[...]
    tnb_ref[...] = jnp.dot(h, wnb_ref[...], preferred_element_type=jnp.float32) + bnb_ref[...]


def _make_tables(h, wof, bof, wnb, bnb):
    return pl.pallas_call(
        _tables_body,
        out_shape=[
            jax.ShapeDtypeStruct((N, 2 * H), jnp.float32),
            jax.ShapeDtypeStruct((N, 2 * H), jnp.float32),
        ],
    )(h, wof, bof, wnb, bnb)


def _make_gather(ptc_of, ptc_nb):
    info = plsc.get_sparse_core_info()
    nc, ns = info.num_cores, info.num_subcores
    nw = nc * ns
    ppad_of = ptc_of * nw * CHUNK
    ppad_nb = ptc_nb * nw * CHUNK
    ptot = ppad_of + ppad_nb
    ptc_max = max(ptc_of, ptc_nb)
    mesh = plsc.VectorSubcoreMesh(core_axis_name="c", subcore_axis_name="s")

    @functools.partial(
        pl.kernel,
        mesh=mesh,
        out_type=jax.ShapeDtypeStruct((ptot, H), jnp.float32),
        scratch_types=[
            pltpu.VMEM((ptc_max, CHUNK), jnp.int32),
            pltpu.VMEM((ptc_max, CHUNK), jnp.int32),
            pltpu.VMEM((NBUF, CHUNK, 2 * H), jnp.float32),
            pltpu.VMEM((NBUF, CHUNK, 2 * H), jnp.float32),
            pltpu.VMEM((NBUF, CHUNK, H), jnp.float32),
            pltpu.SemaphoreType.DMA((NBUF,)),
            pltpu.SemaphoreType.DMA((NBUF,)),
            pltpu.SemaphoreType.DMA((NBUF,)),
        ],
        compiler_params=pltpu.CompilerParams(use_tc_tiling_on_sc=False),
    )
    def k(tof_hbm, i0of_hbm, i1of_hbm, tnb_hbm, i0nb_hbm, i1nb_hbm, s_hbm,
          idx0_v, idx1_v, rows0, rows1, s_buf, gsem0, gsem1, wsem):
        wid = lax.axis_index("s") * nc + lax.axis_index("c")

        def run_level(t_hbm, i0_hbm, i1_hbm, ptc, out_base):
            crow0 = wid * ptc
            pltpu.sync_copy(i0_hbm.at[pl.ds(crow0, ptc)], idx0_v.at[pl.ds(0, ptc)])
            pltpu.sync_copy(i1_hbm.at[pl.ds(crow0, ptc)], idx1_v.at[pl.ds(0, ptc)])

            def fire_g(c, b):
                pltpu.async_copy(t_hbm.at[idx0_v.at[c]], rows0.at[b], gsem0.at[b])
                pltpu.async_copy(t_hbm.at[idx1_v.at[c]], rows1.at[b], gsem1.at[b])

            def wait_g(c, b):
                pltpu.make_async_copy(t_hbm.at[idx0_v.at[c]], rows0.at[b], gsem0.at[b]).wait()
                pltpu.make_async_copy(t_hbm.at[idx1_v.at[c]], rows1.at[b], gsem1.at[b]).wait()

            def compute(b):
                def srow(i, carry):
                    lo0 = rows0[b, i, 0:H]
                    hi0 = rows0[b, i, H:2 * H]
                    lo1 = rows1[b, i, 0:H]
                    hi1 = rows1[b, i, H:2 * H]
                    s_buf[b, i, :] = (jnp.maximum(lo0 + hi1, 0.0)
                                      + jnp.maximum(lo1 + hi0, 0.0))
                    return carry
                lax.fori_loop(0, CHUNK, srow, 0)

            def fire_w(c, b):
                dst = pl.ds(out_base + (crow0 + c) * CHUNK, CHUNK)
                pltpu.async_copy(s_buf.at[b], s_hbm.at[dst], wsem.at[b])

            def wait_w(b):
                pltpu.make_async_copy(s_buf.at[b], s_hbm.at[pl.ds(0, CHUNK)],
                                      wsem.at[b]).wait()

            ngroups = ptc // NBUF
            for b in range(NBUF):
                fire_g(b, b)
            for b in range(NBUF):
                wait_g(b, b)
                compute(b)
                fire_w(b, b)
                fire_g(b + NBUF, b)

            def body(g, carry):
                for b in range(NBUF):
                    c = NBUF * g + b
                    wait_g(c, b)
                    wait_w(b)
                    compute(b)
                    fire_w(c, b)
                    fire_g(c + NBUF, b)
                return carry

            lax.fori_loop(1, ngroups - 1, body, 0)
            for b in range(NBUF):
                c = NBUF * (ngroups - 1) + b
                wait_g(c, b)
                wait_w(b)
                compute(b)
                fire_w(c, b)
            for b in range(NBUF):
                wait_w(b)

        run_level(tof_hbm, i0of_hbm, i1of_hbm, ptc_of, 0)
        run_level(tnb_hbm, i0nb_hbm, i1nb_hbm, ptc_nb, ppad_of)

    return k


GRP = 64
RBLK = 64


def _finish(s2, wb_of, bb_of, wb_nb, bb_nb, ppad_of, ptot):
    blocks_of = ppad_of // (RBLK * GRP)

    def body(s_ref, whof_ref, bhof_ref, whnb_ref, bhnb_ref, out_ref):
        is_of = pl.program_id(0) < blocks_of
        wb = jnp.where(is_of, whof_ref[...], whnb_ref[...])
        bb = jnp.where(is_of, bhof_ref[...], bhnb_ref[...])
        out_ref[...] = jnp.dot(s_ref[...], wb, preferred_element_type=jnp.float32) + bb

    rtot = ptot // GRP
    return pl.pallas_call(
        body,
        grid=(rtot // RBLK,),
        in_specs=[
            pl.BlockSpec((RBLK, GRP * H), lambda i: (i, 0)),
            pl.BlockSpec((GRP * H, 2 * GRP), lambda i: (0, 0)),
            pl.BlockSpec((1, 2 * GRP), lambda i: (0, 0)),
            pl.BlockSpec((GRP * H, 2 * GRP), lambda i: (0, 0)),
            pl.BlockSpec((1, 2 * GRP), lambda i: (0, 0)),
        ],
        out_specs=pl.BlockSpec((RBLK, 2 * GRP), lambda i: (i, 0)),
        out_shape=jax.ShapeDtypeStruct((rtot, 2 * GRP), jnp.float32),
    )(s2, wb_of, bb_of, wb_nb, bb_nb)


def _head_block(wh, bh):
    eye = jnp.eye(GRP, dtype=jnp.float32)
    wb = jnp.einsum("jk,he->jhke", eye, wh).reshape(GRP * H, 2 * GRP)
    bb = jnp.tile(bh, (1, GRP))
    return wb, bb


def _pad_idx(idx, ppad):
    p = idx.shape[0]
    if p != ppad:
        idx = jnp.concatenate([idx, jnp.zeros((ppad - p,), jnp.int32)])
    return idx.reshape(ppad // CHUNK, CHUNK)


def kernel(h, idx0_onefour, idx1_onefour, idx0_nonbonded, idx1_nonbonded,
           W1_of, b1_of, Wsig_of, bsig_of, Weps_of, beps_of,
           W1_nb, b1_nb, Wsig_nb, bsig_nb, Weps_nb, beps_nb):
    wof = jnp.concatenate([W1_of[:D], W1_of[D:]], axis=1)
    wnb = jnp.concatenate([W1_nb[:D], W1_nb[D:]], axis=1)
    bof = jnp.concatenate([jnp.zeros((H,), jnp.float32), b1_of]).reshape(1, 2 * H)
    bnb = jnp.concatenate([jnp.zeros((H,), jnp.float32), b1_nb]).reshape(1, 2 * H)
    wh_of = jnp.concatenate([Wsig_of, Weps_of], axis=1)
    wh_nb = jnp.concatenate([Wsig_nb, Weps_nb], axis=1)
    bh_of = jnp.concatenate([bsig_of, beps_of]).reshape(1, 2)
    bh_nb = jnp.concatenate([bsig_nb, beps_nb]).reshape(1, 2)

    t_of, t_nb = _make_tables(h, wof, bof, wnb, bnb)

    p_of = idx0_onefour.shape[0]
    p_nb = idx0_nonbonded.shape[0]
    gran = 32 * CHUNK * NBUF * 2
    ppad_of = ((p_of + gran - 1) // gran) * gran
    ppad_nb = ((p_nb + gran - 1) // gran) * gran

    s = _make_gather(ppad_of // (32 * CHUNK), ppad_nb // (32 * CHUNK))(
        t_of, _pad_idx(idx0_onefour, ppad_of), _pad_idx(idx1_onefour, ppad_of),
        t_nb, _pad_idx(idx0_nonbonded, ppad_nb), _pad_idx(idx1_nonbonded, ppad_nb))

    ptot = ppad_of + ppad_nb
    wb_of, bb_of = _head_block(wh_of, bh_of)
    wb_nb, bb_nb = _head_block(wh_nb, bh_nb)
    out = _finish(s.reshape(ptot // GRP, GRP * H), wb_of, bb_of, wb_nb, bb_nb,
                  ppad_of, ptot).reshape(ptot, 2)
    return (out[:p_of], out[ppad_of:ppad_of + p_nb])


# speedup vs baseline: 1.2674x; 1.2321x over previous
import functools

import jax
import jax.numpy as jnp
from jax import lax
from jax.experimental import pallas as pl
from jax.experimental.pallas import tpu as pltpu
from jax.experimental.pallas import tpu_sc as plsc

N = 10000
D = 128
H = 16
CHUNK = 128
NBUF = 2


def _tables_body(h_ref, wof_ref, bof_ref, wnb_ref, bnb_ref, tof_ref, tnb_ref):
    h = h_ref[...]
    tof_ref[...] = jnp.dot(h, wof_ref[...], preferred_element_type=jnp.float32) + bof_ref[...]
    tnb_ref[...] = jnp.dot(h, wnb_ref[...], preferred_element_type=jnp.float32) + bnb_ref[...]


def _make_tables(h, wof, bof, wnb, bnb):
    return pl.pallas_call(
        _tables_body,
        out_shape=[
            jax.ShapeDtypeStruct((N, 2 * H), jnp.float32),
            jax.ShapeDtypeStruct((N, 2 * H), jnp.float32),
        ],
    )(h, wof, bof, wnb, bnb)


def _make_sc(ptc_of, ptc_nb):
    info = plsc.get_sparse_core_info()
    nc, ns = info.num_cores, info.num_subcores
    nw = nc * ns
    ppad_of = ptc_of * nw * CHUNK
    ppad_nb = ptc_nb * nw * CHUNK
    ptot = ppad_of + ppad_nb
    ptc_max = max(ptc_of, ptc_nb)
    mesh = plsc.VectorSubcoreMesh(core_axis_name="c", subcore_axis_name="s")

    @functools.partial(
        pl.kernel,
        mesh=mesh,
        out_type=jax.ShapeDtypeStruct((2 * ptot,), jnp.float32),
        scratch_types=[
            pltpu.VMEM((ptc_max, CHUNK), jnp.int32),
            pltpu.VMEM((ptc_max, CHUNK), jnp.int32),
            pltpu.VMEM((NBUF, CHUNK, 2 * H), jnp.float32),
            pltpu.VMEM((NBUF, CHUNK, 2 * H), jnp.float32),
            pltpu.VMEM((NBUF, CHUNK, H), jnp.float32),
            pltpu.VMEM((NBUF, 2 * CHUNK), jnp.float32),
            pltpu.VMEM((4, H), jnp.float32),
            pltpu.SemaphoreType.DMA((NBUF,)),
            pltpu.SemaphoreType.DMA((NBUF,)),
            pltpu.SemaphoreType.DMA((NBUF,)),
        ],
        compiler_params=pltpu.CompilerParams(use_tc_tiling_on_sc=False,
                                            needs_layout_passes=False),
    )
    def k(tof_hbm, i0of_hbm, i1of_hbm, wof_hbm, tnb_hbm, i0nb_hbm, i1nb_hbm,
          wnb_hbm, out_hbm,
          idx0_v, idx1_v, rows0, rows1, s_buf, out_buf, wv, gsem0, gsem1, wsem):
        wid = lax.axis_index("s") * nc + lax.axis_index("c")

        def run_level(t_hbm, i0_hbm, i1_hbm, w_hbm, ptc, out_base):
            crow0 = wid * ptc
            pltpu.sync_copy(i0_hbm.at[pl.ds(crow0, ptc)], idx0_v.at[pl.ds(0, ptc)])
            pltpu.sync_copy(i1_hbm.at[pl.ds(crow0, ptc)], idx1_v.at[pl.ds(0, ptc)])
            pltpu.sync_copy(w_hbm, wv)

            def fire_g(c, b):
                pltpu.async_copy(t_hbm.at[idx0_v.at[c]], rows0.at[b], gsem0.at[b])
                pltpu.async_copy(t_hbm.at[idx1_v.at[c]], rows1.at[b], gsem1.at[b])

            def wait_g(c, b):
                pltpu.make_async_copy(t_hbm.at[idx0_v.at[c]], rows0.at[b], gsem0.at[b]).wait()
                pltpu.make_async_copy(t_hbm.at[idx1_v.at[c]], rows1.at[b], gsem1.at[b]).wait()

            def compute(b):
                def srow(i, carry):
                    lo0 = rows0[b, i, 0:H]
                    hi0 = rows0[b, i, H:2 * H]
                    lo1 = rows1[b, i, 0:H]
                    hi1 = rows1[b, i, H:2 * H]
                    s_buf[b, i, :] = (jnp.maximum(lo0 + hi1, 0.0)
                                      + jnp.maximum(lo1 + hi0, 0.0))
                    return carry
                lax.fori_loop(0, CHUNK, srow, 0)

                iota = lax.iota(jnp.int32, H)
                w0 = wv[0, :]
                w1 = wv[1, :]
                brow = wv[2, :]

                def grp(g, carry):
                    prow = iota + g * H
                    acc0 = jnp.zeros((H,), jnp.float32) + brow[0]
                    acc1 = jnp.zeros((H,), jnp.float32) + brow[1]
                    for j in range(H):
                        col = plsc.load_gather(
                            s_buf.at[b], [prow, jnp.full((H,), j, jnp.int32)])
                        acc0 = acc0 + col * w0[j]
                        acc1 = acc1 + col * w1[j]
                    pos = iota * 2 + g * (2 * H)
                    plsc.store_scatter(out_buf.at[b], [pos], acc0)
                    plsc.store_scatter(out_buf.at[b], [pos + 1], acc1)
                    return carry

                lax.fori_loop(0, CHUNK // H, grp, 0)

            def fire_w(c, b):
                dst = pl.ds(2 * (out_base + (crow0 + c) * CHUNK), 2 * CHUNK)
                pltpu.async_copy(out_buf.at[b], out_hbm.at[dst], wsem.at[b])

            def wait_w(b):
                pltpu.make_async_copy(out_buf.at[b], out_hbm.at[pl.ds(0, 2 * CHUNK)],
                                      wsem.at[b]).wait()

            ngroups = ptc // NBUF
            for b in range(NBUF):
                fire_g(b, b)
            for b in range(NBUF):
                wait_g(b, b)
                compute(b)
                fire_w(b, b)
                fire_g(b + NBUF, b)

            def body(g, carry):
                for b in range(NBUF):
                    c = NBUF * g + b
                    wait_g(c, b)
                    wait_w(b)
                    compute(b)
                    fire_w(c, b)
                    fire_g(c + NBUF, b)
                return carry

            lax.fori_loop(1, ngroups - 1, body, 0)
            for b in range(NBUF):
                c = NBUF * (ngroups - 1) + b
                wait_g(c, b)
                wait_w(b)
                compute(b)
                fire_w(c, b)
            for b in range(NBUF):
                wait_w(b)

        run_level(tof_hbm, i0of_hbm, i1of_hbm, wof_hbm, ptc_of, 0)
        run_level(tnb_hbm, i0nb_hbm, i1nb_hbm, wnb_hbm, ptc_nb, ppad_of)

    return k


def _pad_idx(idx, ppad):
    p = idx.shape[0]
    if p != ppad:
        idx = jnp.concatenate([idx, jnp.zeros((ppad - p,), jnp.int32)])
    return idx.reshape(ppad // CHUNK, CHUNK)


def kernel(h, idx0_onefour, idx1_onefour, idx0_nonbonded, idx1_nonbonded,
           W1_of, b1_of, Wsig_of, bsig_of, Weps_of, beps_of,
           W1_nb, b1_nb, Wsig_nb, bsig_nb, Weps_nb, beps_nb):
    wof = jnp.concatenate([W1_of[:D], W1_of[D:]], axis=1)
    wnb = jnp.concatenate([W1_nb[:D], W1_nb[D:]], axis=1)
    bof = jnp.concatenate([jnp.zeros((H,), jnp.float32), b1_of]).reshape(1, 2 * H)
    bnb = jnp.concatenate([jnp.zeros((H,), jnp.float32), b1_nb]).reshape(1, 2 * H)

    def headpack(wsig, weps, bsig, beps):
        brow = jnp.concatenate([bsig, beps, jnp.zeros((H - 2,), jnp.float32)])
        return jnp.stack([wsig[:, 0], weps[:, 0], brow,
                          jnp.zeros((H,), jnp.float32)])

    wh_of = headpack(Wsig_of, Weps_of, bsig_of, beps_of)
    wh_nb = headpack(Wsig_nb, Weps_nb, bsig_nb, beps_nb)

    t_of, t_nb = _make_tables(h, wof, bof, wnb, bnb)

    p_of = idx0_onefour.shape[0]
    p_nb = idx0_nonbonded.shape[0]
    gran = 32 * CHUNK * NBUF
    ppad_of = ((p_of + gran - 1) // gran) * gran
    ppad_nb = ((p_nb + gran - 1) // gran) * gran
    ptot = ppad_of + ppad_nb

    out1d = _make_sc(ppad_of // (32 * CHUNK), ppad_nb // (32 * CHUNK))(
        t_of, _pad_idx(idx0_onefour, ppad_of), _pad_idx(idx1_onefour, ppad_of),
        wh_of,
        t_nb, _pad_idx(idx0_nonbonded, ppad_nb), _pad_idx(idx1_nonbonded, ppad_nb),
        wh_nb)

    out = out1d.reshape(ptot, 2)
    return (out[:p_of], out[ppad_of:ppad_of + p_nb])


# speedup vs baseline: 1.2704x; 1.0024x over previous
import functools

import jax
import jax.numpy as jnp
from jax import lax
from jax.experimental import pallas as pl
from jax.experimental.pallas import tpu as pltpu
from jax.experimental.pallas import tpu_sc as plsc

N = 10000
D = 128
H = 16
CHUNK = 128
NBUF = 2


def _tables_body(h_ref, wof_ref, bof_ref, wnb_ref, bnb_ref, tof_ref, tnb_ref):
    h = h_ref[...]
    tof_ref[...] = jnp.dot(h, wof_ref[...], preferred_element_type=jnp.float32) + bof_ref[...]
    tnb_ref[...] = jnp.dot(h, wnb_ref[...], preferred_element_type=jnp.float32) + bnb_ref[...]


def _make_tables(h, wof, bof, wnb, bnb):
    return pl.pallas_call(
        _tables_body,
        out_shape=[
            jax.ShapeDtypeStruct((N, 2 * H), jnp.float32),
            jax.ShapeDtypeStruct((N, 2 * H), jnp.float32),
        ],
    )(h, wof, bof, wnb, bnb)


def _make_sc(ptc_of, ptc_nb):
    info = plsc.get_sparse_core_info()
    nc, ns = info.num_cores, info.num_subcores
    nw = nc * ns
    ppad_of = ptc_of * nw * CHUNK
    ppad_nb = ptc_nb * nw * CHUNK
    ptot = ppad_of + ppad_nb
    ptc_max = max(ptc_of, ptc_nb)
    mesh = plsc.VectorSubcoreMesh(core_axis_name="c", subcore_axis_name="s")

    @functools.partial(
        pl.kernel,
        mesh=mesh,
        out_type=jax.ShapeDtypeStruct((2 * ptot,), jnp.float32),
        scratch_types=[
            pltpu.VMEM((ptc_max, CHUNK), jnp.int32),
            pltpu.VMEM((ptc_max, CHUNK), jnp.int32),
            pltpu.VMEM((NBUF, CHUNK, 2 * H), jnp.float32),
            pltpu.VMEM((NBUF, CHUNK, 2 * H), jnp.float32),
            pltpu.VMEM((NBUF, CHUNK, H), jnp.float32),
            pltpu.VMEM((NBUF, 2 * CHUNK), jnp.float32),
            pltpu.VMEM((4, H), jnp.float32),
            pltpu.SemaphoreType.DMA((NBUF,)),
            pltpu.SemaphoreType.DMA((NBUF,)),
            pltpu.SemaphoreType.DMA((NBUF,)),
        ],
        compiler_params=pltpu.CompilerParams(use_tc_tiling_on_sc=False,
                                            needs_layout_passes=False),
    )
    def k(tof_hbm, i0of_hbm, i1of_hbm, wof_hbm, tnb_hbm, i0nb_hbm, i1nb_hbm,
          wnb_hbm, out_hbm,
          idx0_v, idx1_v, rows0, rows1, s_buf, out_buf, wv, gsem0, gsem1, wsem):
        wid = lax.axis_index("s") * nc + lax.axis_index("c")

        def run_level(t_hbm, i0_hbm, i1_hbm, w_hbm, ptc, out_base):
            crow0 = wid * ptc
            pltpu.sync_copy(i0_hbm.at[pl.ds(crow0, ptc)], idx0_v.at[pl.ds(0, ptc)])
            pltpu.sync_copy(i1_hbm.at[pl.ds(crow0, ptc)], idx1_v.at[pl.ds(0, ptc)])
            pltpu.sync_copy(w_hbm, wv)

            def fire_g(c, b):
                pltpu.async_copy(t_hbm.at[idx0_v.at[c]], rows0.at[b], gsem0.at[b])
                pltpu.async_copy(t_hbm.at[idx1_v.at[c]], rows1.at[b], gsem1.at[b])

            def wait_g(c, b):
                pltpu.make_async_copy(t_hbm.at[idx0_v.at[c]], rows0.at[b], gsem0.at[b]).wait()
                pltpu.make_async_copy(t_hbm.at[idx1_v.at[c]], rows1.at[b], gsem1.at[b]).wait()

            def compute(b):
                def srow(i, carry):
                    lo0 = rows0[b, i, 0:H]
                    hi0 = rows0[b, i, H:2 * H]
                    lo1 = rows1[b, i, 0:H]
                    hi1 = rows1[b, i, H:2 * H]
                    s_buf[b, i, :] = (jnp.maximum(lo0 + hi1, 0.0)
                                      + jnp.maximum(lo1 + hi0, 0.0))
                    return carry
                lax.fori_loop(0, CHUNK, srow, 0)

                iota = lax.iota(jnp.int32, H)
                w0 = wv[0, :]
                w1 = wv[1, :]
                brow = wv[2, :]

                def grp(g, carry):
                    prow = iota + g * H
                    acc0 = jnp.zeros((H,), jnp.float32) + brow[0]
                    acc1 = jnp.zeros((H,), jnp.float32) + brow[1]
                    for j in range(H):
                        col = plsc.load_gather(
                            s_buf.at[b], [prow, jnp.full((H,), j, jnp.int32)])
                        acc0 = acc0 + col * w0[j]
                        acc1 = acc1 + col * w1[j]
                    pos = iota * 2 + g * (2 * H)
                    plsc.store_scatter(out_buf.at[b], [pos], acc0)
                    plsc.store_scatter(out_buf.at[b], [pos + 1], acc1)
                    return carry

                lax.fori_loop(0, CHUNK // H, grp, 0)

            def fire_w(c, b):
                dst = pl.ds(2 * (out_base + (crow0 + c) * CHUNK), 2 * CHUNK)
                pltpu.async_copy(out_buf.at[b], out_hbm.at[dst], wsem.at[b])

            def wait_w(b):
                pltpu.make_async_copy(out_buf.at[b], out_hbm.at[pl.ds(0, 2 * CHUNK)],
                                      wsem.at[b]).wait()

            ngroups = ptc // NBUF
            for b in range(NBUF):
                fire_g(b, b)
            for b in range(NBUF):
                wait_g(b, b)
                compute(b)
                fire_w(b, b)
                fire_g(b + NBUF, b)

            def body(g, carry):
                for b in range(NBUF):
                    c = NBUF * g + b
                    wait_g(c, b)
                    wait_w(b)
                    compute(b)
                    fire_w(c, b)
                    fire_g(c + NBUF, b)
                return carry

            lax.fori_loop(1, ngroups - 1, body, 0)
            for b in range(NBUF):
                c = NBUF * (ngroups - 1) + b
                wait_g(c, b)
                wait_w(b)
                compute(b)
                fire_w(c, b)
            for b in range(NBUF):
                wait_w(b)

        run_level(tof_hbm, i0of_hbm, i1of_hbm, wof_hbm, ptc_of, 0)
        run_level(tnb_hbm, i0nb_hbm, i1nb_hbm, wnb_hbm, ptc_nb, ppad_of)

    return k


def _pad_idx(idx, ppad):
    p = idx.shape[0]
    if p != ppad:
        idx = jnp.concatenate([idx, jnp.zeros((ppad - p,), jnp.int32)])
    return idx.reshape(ppad // CHUNK, CHUNK)


def kernel(h, idx0_onefour, idx1_onefour, idx0_nonbonded, idx1_nonbonded,
           W1_of, b1_of, Wsig_of, bsig_of, Weps_of, beps_of,
           W1_nb, b1_nb, Wsig_nb, bsig_nb, Weps_nb, beps_nb):
    wof = jnp.concatenate([W1_of[:D], W1_of[D:]], axis=1)
    wnb = jnp.concatenate([W1_nb[:D], W1_nb[D:]], axis=1)
    bof = jnp.concatenate([jnp.zeros((H,), jnp.float32), b1_of]).reshape(1, 2 * H)
    bnb = jnp.concatenate([jnp.zeros((H,), jnp.float32), b1_nb]).reshape(1, 2 * H)

    def headpack(wsig, weps, bsig, beps):
        brow = jnp.concatenate([bsig, beps, jnp.zeros((H - 2,), jnp.float32)])
        return jnp.stack([wsig[:, 0], weps[:, 0], brow,
                          jnp.zeros((H,), jnp.float32)])

    wh_of = headpack(Wsig_of, Weps_of, bsig_of, beps_of)
    wh_nb = headpack(Wsig_nb, Weps_nb, bsig_nb, beps_nb)

    t_of, t_nb = _make_tables(h, wof, bof, wnb, bnb)

    p_of = idx0_onefour.shape[0]
    p_nb = idx0_nonbonded.shape[0]
    gran = 32 * CHUNK * NBUF
    ppad_of = ((p_of + gran - 1) // gran) * gran
    ppad_nb = ((p_nb + gran - 1) // gran) * gran
    ptot = ppad_of + ppad_nb

    out1d = _make_sc(ppad_of // (32 * CHUNK), ppad_nb // (32 * CHUNK))(
        t_of, _pad_idx(idx0_onefour, ppad_of), _pad_idx(idx1_onefour, ppad_of),
        wh_of,
        t_nb, _pad_idx(idx0_nonbonded, ppad_nb), _pad_idx(idx1_nonbonded, ppad_nb),
        wh_nb)

    out_of = out1d[:2 * p_of].reshape(p_of, 2)
    out_nb = out1d[2 * ppad_of:2 * (ppad_of + p_nb)].reshape(p_nb, 2)
    return (out_of, out_nb)


# speedup vs baseline: 1.8605x; 1.4644x over previous
import functools

import jax
import jax.numpy as jnp
from jax import lax
from jax.experimental import pallas as pl
from jax.experimental.pallas import tpu as pltpu
from jax.experimental.pallas import tpu_sc as plsc

N = 10000
D = 128
H = 16
CHUNK = 128
NBUF = 2


def _tables_body(h_ref, wof_ref, bof_ref, wnb_ref, bnb_ref, tof_ref, tnb_ref):
    h = h_ref[...]
    tof_ref[...] = jnp.dot(h, wof_ref[...], preferred_element_type=jnp.float32) + bof_ref[...]
    tnb_ref[...] = jnp.dot(h, wnb_ref[...], preferred_element_type=jnp.float32) + bnb_ref[...]


def _make_tables(h, wof, bof, wnb, bnb):
    return pl.pallas_call(
        _tables_body,
        out_shape=[
            jax.ShapeDtypeStruct((N, 2 * H), jnp.float32),
            jax.ShapeDtypeStruct((N, 2 * H), jnp.float32),
        ],
    )(h, wof, bof, wnb, bnb)


def _make_sc(ptc_of, ptc_nb):
    info = plsc.get_sparse_core_info()
    nc, ns = info.num_cores, info.num_subcores
    nw = nc * ns
    ppad_of = ptc_of * nw * CHUNK
    ppad_nb = ptc_nb * nw * CHUNK
    ptot = ppad_of + ppad_nb
    ptc_max = max(ptc_of, ptc_nb)
    mesh = plsc.VectorSubcoreMesh(core_axis_name="c", subcore_axis_name="s")

    @functools.partial(
        pl.kernel,
        mesh=mesh,
        out_type=jax.ShapeDtypeStruct((2 * ptot,), jnp.float32),
        scratch_types=[
            pltpu.VMEM_SHARED((N, 2 * H), jnp.float32),
            pltpu.VMEM_SHARED((N, 2 * H), jnp.float32),
            pltpu.VMEM((ptc_max, CHUNK), jnp.int32),
            pltpu.VMEM((ptc_max, CHUNK), jnp.int32),
            pltpu.VMEM((NBUF, CHUNK, 2 * H), jnp.float32),
            pltpu.VMEM((NBUF, CHUNK, 2 * H), jnp.float32),
            pltpu.VMEM((NBUF, CHUNK, H), jnp.float32),
            pltpu.VMEM((NBUF, 2 * CHUNK), jnp.float32),
            pltpu.VMEM((4, H), jnp.float32),
            pltpu.SemaphoreType.DMA((NBUF,)),
            pltpu.SemaphoreType.DMA((NBUF,)),
            pltpu.SemaphoreType.DMA((NBUF,)),
        ],
        compiler_params=pltpu.CompilerParams(use_tc_tiling_on_sc=False,
                                            needs_layout_passes=False),
    )
    def k(tof_hbm, i0of_hbm, i1of_hbm, wof_hbm, tnb_hbm, i0nb_hbm, i1nb_hbm,
          wnb_hbm, out_hbm,
          tof_sh, tnb_sh,
          idx0_v, idx1_v, rows0, rows1, s_buf, out_buf, wv, gsem0, gsem1, wsem):
        wid = lax.axis_index("s") * nc + lax.axis_index("c")
        sid = lax.axis_index("s")

        rows_per = N // ns
        stripe = pl.ds(sid * rows_per, rows_per)
        pltpu.sync_copy(tof_hbm.at[stripe], tof_sh.at[stripe])
        pltpu.sync_copy(tnb_hbm.at[stripe], tnb_sh.at[stripe])
        plsc.subcore_barrier()

        def run_level(t_hbm, i0_hbm, i1_hbm, w_hbm, ptc, out_base):
            crow0 = wid * ptc
            pltpu.sync_copy(i0_hbm.at[pl.ds(crow0, ptc)], idx0_v.at[pl.ds(0, ptc)])
            pltpu.sync_copy(i1_hbm.at[pl.ds(crow0, ptc)], idx1_v.at[pl.ds(0, ptc)])
            pltpu.sync_copy(w_hbm, wv)

            def fire_g(c, b):
                pltpu.async_copy(t_hbm.at[idx0_v.at[c]], rows0.at[b], gsem0.at[b])
                pltpu.async_copy(t_hbm.at[idx1_v.at[c]], rows1.at[b], gsem1.at[b])

            def wait_g(c, b):
                pltpu.make_async_copy(t_hbm.at[idx0_v.at[c]], rows0.at[b], gsem0.at[b]).wait()
                pltpu.make_async_copy(t_hbm.at[idx1_v.at[c]], rows1.at[b], gsem1.at[b]).wait()

            def compute(b):
                def srow(i, carry):
                    lo0 = rows0[b, i, 0:H]
                    hi0 = rows0[b, i, H:2 * H]
                    lo1 = rows1[b, i, 0:H]
                    hi1 = rows1[b, i, H:2 * H]
                    s_buf[b, i, :] = (jnp.maximum(lo0 + hi1, 0.0)
                                      + jnp.maximum(lo1 + hi0, 0.0))
                    return carry
                lax.fori_loop(0, CHUNK, srow, 0)

                iota = lax.iota(jnp.int32, H)
                w0 = wv[0, :]
                w1 = wv[1, :]
                brow = wv[2, :]

                def grp(g, carry):
                    prow = iota + g * H
                    acc0 = jnp.zeros((H,), jnp.float32) + brow[0]
                    acc1 = jnp.zeros((H,), jnp.float32) + brow[1]
                    for j in range(H):
                        col = plsc.load_gather(
                            s_buf.at[b], [prow, jnp.full((H,), j, jnp.int32)])
                        acc0 = acc0 + col * w0[j]
                        acc1 = acc1 + col * w1[j]
                    pos = iota * 2 + g * (2 * H)
                    plsc.store_scatter(out_buf.at[b], [pos], acc0)
                    plsc.store_scatter(out_buf.at[b], [pos + 1], acc1)
                    return carry

                lax.fori_loop(0, CHUNK // H, grp, 0)

            def fire_w(c, b):
                dst = pl.ds(2 * (out_base + (crow0 + c) * CHUNK), 2 * CHUNK)
                pltpu.async_copy(out_buf.at[b], out_hbm.at[dst], wsem.at[b])

            def wait_w(b):
                pltpu.make_async_copy(out_buf.at[b], out_hbm.at[pl.ds(0, 2 * CHUNK)],
                                      wsem.at[b]).wait()

            ngroups = ptc // NBUF
            for b in range(NBUF):
                fire_g(b, b)
            for b in range(NBUF):
                wait_g(b, b)
                compute(b)
                fire_w(b, b)
                fire_g(b + NBUF, b)

            def body(g, carry):
                for b in range(NBUF):
                    c = NBUF * g + b
                    wait_g(c, b)
                    wait_w(b)
                    compute(b)
                    fire_w(c, b)
                    fire_g(c + NBUF, b)
                return carry

            lax.fori_loop(1, ngroups - 1, body, 0)
            for b in range(NBUF):
                c = NBUF * (ngroups - 1) + b
                wait_g(c, b)
                wait_w(b)
                compute(b)
                fire_w(c, b)
            for b in range(NBUF):
                wait_w(b)

        run_level(tof_sh, i0of_hbm, i1of_hbm, wof_hbm, ptc_of, 0)
        run_level(tnb_sh, i0nb_hbm, i1nb_hbm, wnb_hbm, ptc_nb, ppad_of)

    return k


def _pad_idx(idx, ppad):
    p = idx.shape[0]
    if p != ppad:
        idx = jnp.concatenate([idx, jnp.zeros((ppad - p,), jnp.int32)])
    return idx.reshape(ppad // CHUNK, CHUNK)


def kernel(h, idx0_onefour, idx1_onefour, idx0_nonbonded, idx1_nonbonded,
           W1_of, b1_of, Wsig_of, bsig_of, Weps_of, beps_of,
           W1_nb, b1_nb, Wsig_nb, bsig_nb, Weps_nb, beps_nb):
    wof = jnp.concatenate([W1_of[:D], W1_of[D:]], axis=1)
    wnb = jnp.concatenate([W1_nb[:D], W1_nb[D:]], axis=1)
    bof = jnp.concatenate([jnp.zeros((H,), jnp.float32), b1_of]).reshape(1, 2 * H)
    bnb = jnp.concatenate([jnp.zeros((H,), jnp.float32), b1_nb]).reshape(1, 2 * H)

    def headpack(wsig, weps, bsig, beps):
        brow = jnp.concatenate([bsig, beps, jnp.zeros((H - 2,), jnp.float32)])
        return jnp.stack([wsig[:, 0], weps[:, 0], brow,
                          jnp.zeros((H,), jnp.float32)])

    wh_of = headpack(Wsig_of, Weps_of, bsig_of, beps_of)
    wh_nb = headpack(Wsig_nb, Weps_nb, bsig_nb, beps_nb)

    t_of, t_nb = _make_tables(h, wof, bof, wnb, bnb)

    p_of = idx0_onefour.shape[0]
    p_nb = idx0_nonbonded.shape[0]
    gran = 32 * CHUNK * NBUF
    ppad_of = ((p_of + gran - 1) // gran) * gran
    ppad_nb = ((p_nb + gran - 1) // gran) * gran
    ptot = ppad_of + ppad_nb

    out1d = _make_sc(ppad_of // (32 * CHUNK), ppad_nb // (32 * CHUNK))(
        t_of, _pad_idx(idx0_onefour, ppad_of), _pad_idx(idx1_onefour, ppad_of),
        wh_of,
        t_nb, _pad_idx(idx0_nonbonded, ppad_nb), _pad_idx(idx1_nonbonded, ppad_nb),
        wh_nb)

    out_of = out1d[:2 * p_of].reshape(p_of, 2)
    out_nb = out1d[2 * ppad_of:2 * (ppad_of + p_nb)].reshape(p_nb, 2)
    return (out_of, out_nb)
